# baseline (speedup 1.0000x reference)
import functools

import jax
import jax.numpy as jnp
from jax import lax
from jax.experimental import pallas as pl
from jax.experimental.pallas import tpu as pltpu
from jax.experimental.pallas import tpu_sc as plsc

N, E, D, H = 10000, 160000, 256, 64
NC, NS = 2, 16
NW = NC * NS
N_PAD = 10240
DROWS = N_PAD // 128
DRPT = DROWS // NS
E_PAD = 163840
EPT = E_PAD // NW
K = 80
NCHUNK = EPT // K
NBUF = 4
RPT = N_PAD // NS
MBLK = 1024
NBLK = N_PAD // MBLK

_mesh = plsc.VectorSubcoreMesh(core_axis_name="c", subcore_axis_name="s")


def _rsqrt_newton(d):
    i = plsc.bitcast(d, jnp.int32)
    i = 0x5F3759DF - lax.shift_right_logical(i, 1)
    y = plsc.bitcast(i, jnp.float32)
    half = -0.5 * d
    for _ in range(3):
        y = y * (1.5 + half * y * y)
    return y


@functools.partial(
    pl.kernel,
    out_type=jax.ShapeDtypeStruct((NC, DROWS, 128), jnp.float32),
    mesh=_mesh,
    scratch_types=[
        pltpu.VMEM((NCHUNK, K), jnp.int32),
        pltpu.VMEM((NCHUNK, K), jnp.float32),
        pltpu.VMEM((DROWS, 128), jnp.float32),
        pltpu.VMEM((DROWS,), jnp.int32),
        pltpu.VMEM_SHARED((DROWS, 128), jnp.float32),
    ],
    compiler_params=pltpu.CompilerParams(needs_layout_passes=False,
                                         use_tc_tiling_on_sc=False),
)
def _deg_kernel(ei_hbm, w_hbm, out_hbm, colv, wv, degv, idxv, dacc):
    cid = lax.axis_index("c")
    sid = lax.axis_index("s")
    wid = cid * NS + sid
    pltpu.sync_copy(ei_hbm.at[1, wid], colv)
    pltpu.sync_copy(w_hbm.at[wid], wv)

    zeros = jnp.zeros((16,), jnp.float32)

    def zbody(i, carry):
        for j in range(8):
            degv[i, pl.ds(j * 16, 16)] = zeros
        return carry

    lax.fori_loop(0, DROWS, zbody, 0)

    for i in range(DROWS // 16):
        idxv[pl.ds(i * 16, 16)] = lax.iota(jnp.int32, 16) + (i * 16)

    pltpu.sync_copy(degv.at[pl.ds(sid * DRPT, DRPT)],
                    dacc.at[pl.ds(sid * DRPT, DRPT)])
    plsc.subcore_barrier()

    def ebody(ci, carry):
        for i in range(K // 16):
            sl = pl.ds(i * 16, 16)
            c = colv[ci, sl]
            v = wv[ci, sl]
            hi = lax.shift_right_logical(c, 7)
            lo = lax.bitwise_and(c, 127)
            plsc.addupdate_scatter(degv, [hi, lo], v)
        return carry

    lax.fori_loop(0, NCHUNK, ebody, 0)

    pltpu.sync_copy(degv, dacc.at[idxv], add=True)
    plsc.subcore_barrier()
    pltpu.sync_copy(dacc.at[pl.ds(sid * DRPT, DRPT)],
                    out_hbm.at[cid, pl.ds(sid * DRPT, DRPT)])


@functools.partial(
    pl.kernel,
    out_type=jax.ShapeDtypeStruct((N_PAD, 2 * H), jnp.float32),
    mesh=_mesh,
    scratch_types=[
        pltpu.VMEM((NCHUNK, K), jnp.int32),
        pltpu.VMEM((NCHUNK, K), jnp.int32),
        pltpu.VMEM((NCHUNK, K), jnp.float32),
        pltpu.VMEM((N_PAD,), jnp.float32),
        pltpu.VMEM((DRPT, 128), jnp.float32),
        pltpu.VMEM((DRPT, 128), jnp.float32),
        pltpu.VMEM((K, H), jnp.float32),
        pltpu.VMEM((K, H), jnp.float32),
        pltpu.VMEM((K, H), jnp.float32),
        pltpu.VMEM((K, H), jnp.float32),
        pltpu.VMEM_SHARED((N_PAD, H), jnp.float32),
        pltpu.VMEM_SHARED((N_PAD, H), jnp.float32),
        pltpu.VMEM_SHARED((N_PAD,), jnp.float32),
        pltpu.SemaphoreType.DMA,
        pltpu.SemaphoreType.DMA,
        pltpu.SemaphoreType.DMA,
        pltpu.SemaphoreType.DMA,
        pltpu.SemaphoreType.DMA,
        pltpu.SemaphoreType.DMA,
        pltpu.SemaphoreType.DMA,
        pltpu.SemaphoreType.DMA,
    ],
    compiler_params=pltpu.CompilerParams(needs_layout_passes=False,
                                         use_tc_tiling_on_sc=False),
)
def _msg_kernel(h_hbm, deg_hbm, ei_hbm, w_hbm, out_hbm,
                ridx, cidx, wv, dinv_v, da, db, bb0, bb1, bb2, bb3,
                accum, hsh, dsh,
                gsem0, gsem1, gsem2, gsem3, ssem0, ssem1, ssem2, ssem3):
    cid = lax.axis_index("c")
    sid = lax.axis_index("s")
    wid = cid * NS + sid
    buf = (bb0, bb1, bb2, bb3)
    gsem = (gsem0, gsem1, gsem2, gsem3)
    ssem = (ssem0, ssem1, ssem2, ssem3)

    pltpu.sync_copy(ei_hbm.at[0, wid], ridx)
    pltpu.sync_copy(ei_hbm.at[1, wid], cidx)
    pltpu.sync_copy(w_hbm.at[wid], wv)

    stripe = pl.ds(sid * RPT, RPT)
    pltpu.sync_copy(deg_hbm.at[0, pl.ds(sid * DRPT, DRPT)], da)
    pltpu.sync_copy(deg_hbm.at[1, pl.ds(sid * DRPT, DRPT)], db)

    for r in range(DRPT):
        for j in range(8):
            sl = pl.ds(j * 16, 16)
            d = da[r, sl] + db[r, sl] + 1.0
            dinv_v[pl.ds((r * 8 + j) * 16, 16)] = _rsqrt_newton(d)

    pltpu.sync_copy(dinv_v.at[pl.ds(0, RPT)], dsh.at[stripe])

    zeros = jnp.zeros((16,), jnp.float32)

    def zbody(i, carry):
        for j in range(H // 16):
            bb0[i, pl.ds(j * 16, 16)] = zeros
        return carry

    lax.fori_loop(0, K, zbody, 0)

    def zcopy(i, carry):
        pltpu.sync_copy(bb0, accum.at[pl.ds(sid * RPT + i * K, K)])
        return carry

    lax.fori_loop(0, RPT // K, zcopy, 0)

    pltpu.sync_copy(h_hbm.at[stripe, pl.ds(0, H)], hsh.at[stripe])
    plsc.subcore_barrier()

    pltpu.sync_copy(dsh, dinv_v)

    pltpu.async_copy(hsh.at[ridx.at[0]], bb0, gsem0)
    pltpu.async_copy(hsh.at[ridx.at[1]], bb1, gsem1)

    def nbody(ci, carry):
        for i in range(K // 16):
            sl = pl.ds(i * 16, 16)
            wv[ci, sl] = wv[ci, sl] * plsc.load_gather(dinv_v, [ridx[ci, sl]])
        return carry

    lax.fori_loop(0, NCHUNK, nbody, 0)

    def quad(t, carry):
        for b in range(NBUF):
            ci = NBUF * t + b
            bn = (b + 2) % NBUF

            @pl.when((ci >= 2) & (ci + 2 < NCHUNK))
            def _wait_scatter():
                pltpu.make_async_copy(buf[bn], accum.at[cidx.at[0]],
                                      ssem[bn]).wait()

            @pl.when(ci + 2 < NCHUNK)
            def _next_gather():
                pltpu.async_copy(hsh.at[ridx.at[ci + 2]],
                                 buf[bn], gsem[bn])

            pltpu.make_async_copy(hsh.at[ridx.at[0]], buf[b],
                                  gsem[b]).wait()
            civ = lax.broadcast(ci, (16,))

            @plsc.parallel_loop(0, K, 1, unroll=16)
            def scale(k):
                ws = plsc.load_gather(wv, [civ, lax.broadcast(k, (16,))])
                for j in range(H // 16):
                    sl = pl.ds(j * 16, 16)
                    buf[b][k, sl] = buf[b][k, sl] * ws

            pltpu.async_copy(buf[b], accum.at[cidx.at[ci]], ssem[b], add=True)

        return carry

    lax.fori_loop(0, NCHUNK // NBUF, quad, 0)

    for b in range(NBUF):
        pltpu.make_async_copy(buf[b], accum.at[cidx.at[0]], ssem[b]).wait()
    plsc.subcore_barrier()
    pltpu.sync_copy(accum.at[stripe], out_hbm.at[stripe, pl.ds(cid * H, H)])


def _b_body(x_ref, w1_ref, h1_ref):
    hmat = jnp.dot(x_ref[...], w1_ref[...], preferred_element_type=jnp.float32)
    h1_ref[...] = jnp.concatenate([hmat, jnp.zeros_like(hmat)], axis=1)


def _dinv_block(deg_ref):
    deg = 1.0 + jnp.sum(deg_ref[...], axis=0)
    return lax.rsqrt(deg)[:, None]


def _d_body(p_ref, h1_ref, deg_ref, b1_ref, w2_ref, h2_ref):
    dinv = _dinv_block(deg_ref)
    p0 = p_ref[:, :H]
    p1 = p_ref[:, H:]
    t = (p0 + p1) * dinv + h1_ref[:, :H] * (dinv * dinv)
    o1 = jnp.maximum(t + b1_ref[...], 0.0)
    hmat = jnp.dot(o1, w2_ref[...], preferred_element_type=jnp.float32)
    h2_ref[...] = jnp.concatenate([hmat, jnp.zeros_like(hmat)], axis=1)


def _f_body(q_ref, h2_ref, deg_ref, b2_ref, out_ref):
    dinv = _dinv_block(deg_ref)
    out_ref[...] = ((q_ref[:, :H] + q_ref[:, H:]) * dinv
                    + h2_ref[:, :H] * (dinv * dinv) + b2_ref[...])


def _tc_b(x, W1):
    return pl.pallas_call(
        _b_body,
        grid=(NBLK,),
        in_specs=[
            pl.BlockSpec((MBLK, D), lambda i: (i, 0)),
            pl.BlockSpec((D, H), lambda i: (0, 0)),
        ],
        out_specs=pl.BlockSpec((MBLK, 2 * H), lambda i: (i, 0)),
        out_shape=jax.ShapeDtypeStruct((N_PAD, 2 * H), jnp.float32),
    )(x, W1)


def _tc_d(p, h1, deg, b1, W2):
    return pl.pallas_call(
        _d_body,
        grid=(NBLK,),
        in_specs=[
            pl.BlockSpec((MBLK, 2 * H), lambda i: (i, 0)),
            pl.BlockSpec((MBLK, 2 * H), lambda i: (i, 0)),
            pl.BlockSpec((NC, MBLK), lambda i: (0, i)),
            pl.BlockSpec((1, H), lambda i: (0, 0)),
            pl.BlockSpec((H, H), lambda i: (0, 0)),
        ],
        out_specs=pl.BlockSpec((MBLK, 2 * H), lambda i: (i, 0)),
        out_shape=jax.ShapeDtypeStruct((N_PAD, 2 * H), jnp.float32),
    )(p, h1, deg, b1, W2)


def _tc_f(q, h2, deg, b2):
    return pl.pallas_call(
        _f_body,
        grid=(NBLK,),
        in_specs=[
            pl.BlockSpec((MBLK, 2 * H), lambda i: (i, 0)),
            pl.BlockSpec((MBLK, 2 * H), lambda i: (i, 0)),
            pl.BlockSpec((NC, MBLK), lambda i: (0, i)),
            pl.BlockSpec((1, H), lambda i: (0, 0)),
        ],
        out_specs=pl.BlockSpec((MBLK, H), lambda i: (i, 0)),
        out_shape=jax.ShapeDtypeStruct((N, H), jnp.float32),
    )(q, h2, deg, b2)


def kernel(x, edge_index, w, W1, b1, W2, b2):
    eip = jnp.pad(edge_index,
                  ((0, 0), (0, E_PAD - E))).reshape(2, NW, NCHUNK, K)
    wp = jnp.pad(w, (0, E_PAD - E)).reshape(NW, NCHUNK, K)

    deg = _deg_kernel(eip, wp)
    h1 = _tc_b(x, W1)
    p = _msg_kernel(h1, deg, eip, wp)
    deg2 = deg.reshape(NC, N_PAD)
    h2 = _tc_d(p, h1, deg2, b1.reshape(1, H), W2)
    q = _msg_kernel(h2, deg, eip, wp)
    return _tc_f(q, h2, deg2, b2.reshape(1, H))


# speedup vs baseline: 1.0117x; 1.0117x over previous
import functools

import jax
import jax.numpy as jnp
from jax import lax
from jax.experimental import pallas as pl
from jax.experimental.pallas import tpu as pltpu
from jax.experimental.pallas import tpu_sc as plsc

N, E, D, H = 10000, 160000, 256, 64
NC, NS = 2, 16
NW = NC * NS
N_PAD = 10240
DROWS = N_PAD // 128
DRPT = DROWS // NS
E_PAD = 163840
EPT = E_PAD // NW
K = 80
NCHUNK = EPT // K
NBUF = 4
RPT = N_PAD // NS
MBLK = 1024
NBLK = N_PAD // MBLK

_mesh = plsc.VectorSubcoreMesh(core_axis_name="c", subcore_axis_name="s")


def _rsqrt_newton(d):
    i = plsc.bitcast(d, jnp.int32)
    i = 0x5F3759DF - lax.shift_right_logical(i, 1)
    y = plsc.bitcast(i, jnp.float32)
    half = -0.5 * d
    for _ in range(3):
        y = y * (1.5 + half * y * y)
    return y


@functools.partial(
    pl.kernel,
    out_type=jax.ShapeDtypeStruct((NC, DROWS, 128), jnp.float32),
    mesh=_mesh,
    scratch_types=[
        pltpu.VMEM((NCHUNK, K), jnp.int32),
        pltpu.VMEM((NCHUNK, K), jnp.float32),
        pltpu.VMEM((DROWS, 128), jnp.float32),
        pltpu.VMEM((DROWS,), jnp.int32),
        pltpu.VMEM_SHARED((DROWS, 128), jnp.float32),
    ],
    compiler_params=pltpu.CompilerParams(needs_layout_passes=False,
                                         use_tc_tiling_on_sc=False),
)
def _deg_kernel(ei_hbm, w_hbm, out_hbm, colv, wv, degv, idxv, dacc):
    cid = lax.axis_index("c")
    sid = lax.axis_index("s")
    wid = cid * NS + sid
    pltpu.sync_copy(ei_hbm.at[1, wid], colv)
    pltpu.sync_copy(w_hbm.at[wid], wv)

    zeros = jnp.zeros((16,), jnp.float32)

    def zbody(i, carry):
        for j in range(8):
            degv[i, pl.ds(j * 16, 16)] = zeros
        return carry

    lax.fori_loop(0, DROWS, zbody, 0)

    for i in range(DROWS // 16):
        idxv[pl.ds(i * 16, 16)] = lax.iota(jnp.int32, 16) + (i * 16)

    pltpu.sync_copy(degv.at[pl.ds(sid * DRPT, DRPT)],
                    dacc.at[pl.ds(sid * DRPT, DRPT)])
    plsc.subcore_barrier()

    def ebody(ci, carry):
        for i in range(K // 16):
            sl = pl.ds(i * 16, 16)
            c = colv[ci, sl]
            v = wv[ci, sl]
            hi = lax.shift_right_logical(c, 7)
            lo = lax.bitwise_and(c, 127)
            plsc.addupdate_scatter(degv, [hi, lo], v)
        return carry

    lax.fori_loop(0, NCHUNK, ebody, 0)

    pltpu.sync_copy(degv, dacc.at[idxv], add=True)
    plsc.subcore_barrier()
    pltpu.sync_copy(dacc.at[pl.ds(sid * DRPT, DRPT)],
                    out_hbm.at[cid, pl.ds(sid * DRPT, DRPT)])


@functools.partial(
    pl.kernel,
    out_type=jax.ShapeDtypeStruct((N_PAD, 2 * H), jnp.float32),
    mesh=_mesh,
    scratch_types=[
        pltpu.VMEM((NCHUNK, K), jnp.int32),
        pltpu.VMEM((NCHUNK, K), jnp.int32),
        pltpu.VMEM((NCHUNK, K), jnp.float32),
        pltpu.VMEM((N_PAD,), jnp.float32),
        pltpu.VMEM((DRPT, 128), jnp.float32),
        pltpu.VMEM((DRPT, 128), jnp.float32),
        pltpu.VMEM((K, H), jnp.float32),
        pltpu.VMEM((K, H), jnp.float32),
        pltpu.VMEM((K, H), jnp.float32),
        pltpu.VMEM((K, H), jnp.float32),
        pltpu.VMEM_SHARED((N_PAD, H), jnp.float32),
        pltpu.VMEM_SHARED((N_PAD, H), jnp.float32),
        pltpu.VMEM_SHARED((N_PAD,), jnp.float32),
        pltpu.SemaphoreType.DMA,
        pltpu.SemaphoreType.DMA,
        pltpu.SemaphoreType.DMA,
        pltpu.SemaphoreType.DMA,
        pltpu.SemaphoreType.DMA,
        pltpu.SemaphoreType.DMA,
        pltpu.SemaphoreType.DMA,
        pltpu.SemaphoreType.DMA,
    ],
    compiler_params=pltpu.CompilerParams(needs_layout_passes=False,
                                         use_tc_tiling_on_sc=False),
)
def _msg_kernel(h_hbm, deg_hbm, ei_hbm, w_hbm, out_hbm,
                ridx, cidx, wv, dinv_v, da, db, bb0, bb1, bb2, bb3,
                accum, hsh, dsh,
                gsem0, gsem1, gsem2, gsem3, ssem0, ssem1, ssem2, ssem3):
    cid = lax.axis_index("c")
    sid = lax.axis_index("s")
    wid = cid * NS + sid
    buf = (bb0, bb1, bb2, bb3)
    gsem = (gsem0, gsem1, gsem2, gsem3)
    ssem = (ssem0, ssem1, ssem2, ssem3)

    pltpu.sync_copy(ei_hbm.at[0, wid], ridx)
    pltpu.sync_copy(ei_hbm.at[1, wid], cidx)
    pltpu.sync_copy(w_hbm.at[wid], wv)

    stripe = pl.ds(sid * RPT, RPT)
    pltpu.sync_copy(deg_hbm.at[0, pl.ds(sid * DRPT, DRPT)], da)
    pltpu.sync_copy(deg_hbm.at[1, pl.ds(sid * DRPT, DRPT)], db)

    for r in range(DRPT):
        for j in range(8):
            sl = pl.ds(j * 16, 16)
            d = da[r, sl] + db[r, sl] + 1.0
            dinv_v[pl.ds((r * 8 + j) * 16, 16)] = _rsqrt_newton(d)

    pltpu.sync_copy(dinv_v.at[pl.ds(0, RPT)], dsh.at[stripe])

    zeros = jnp.zeros((16,), jnp.float32)

    def zbody(i, carry):
        for j in range(H // 16):
            bb0[i, pl.ds(j * 16, 16)] = zeros
        return carry

    lax.fori_loop(0, K, zbody, 0)

    def zcopy(i, carry):
        pltpu.sync_copy(bb0, accum.at[pl.ds(sid * RPT + i * K, K)])
        return carry

    lax.fori_loop(0, RPT // K, zcopy, 0)

    pltpu.sync_copy(h_hbm.at[stripe, pl.ds(0, H)], hsh.at[stripe])
    plsc.subcore_barrier()

    pltpu.sync_copy(dsh, dinv_v)

    pltpu.async_copy(hsh.at[ridx.at[0]], bb0, gsem0)
    pltpu.async_copy(hsh.at[ridx.at[1]], bb1, gsem1)

    def nbody(ci, carry):
        for i in range(K // 16):
            sl = pl.ds(i * 16, 16)
            wv[ci, sl] = wv[ci, sl] * plsc.load_gather(dinv_v, [ridx[ci, sl]])
        return carry

    lax.fori_loop(0, NCHUNK, nbody, 0)

    def quad(t, carry):
        for b in range(NBUF):
            ci = NBUF * t + b
            bn = (b + 2) % NBUF

            @pl.when((ci >= 2) & (ci + 2 < NCHUNK))
            def _wait_scatter():
                pltpu.make_async_copy(buf[bn], accum.at[cidx.at[0]],
                                      ssem[bn]).wait()

            @pl.when(ci + 2 < NCHUNK)
            def _next_gather():
                pltpu.async_copy(hsh.at[ridx.at[ci + 2]],
                                 buf[bn], gsem[bn])

            pltpu.make_async_copy(hsh.at[ridx.at[0]], buf[b],
                                  gsem[b]).wait()
            civ = lax.broadcast(ci, (16,))

            @plsc.parallel_loop(0, K, 1, unroll=8)
            def scale(k):
                ws = plsc.load_gather(wv, [civ, lax.broadcast(k, (16,))])
                for j in range(H // 16):
                    sl = pl.ds(j * 16, 16)
                    buf[b][k, sl] = buf[b][k, sl] * ws

            pltpu.async_copy(buf[b], accum.at[cidx.at[ci]], ssem[b], add=True)

        return carry

    lax.fori_loop(0, NCHUNK // NBUF, quad, 0)

    for b in range(NBUF):
        pltpu.make_async_copy(buf[b], accum.at[cidx.at[0]], ssem[b]).wait()
    plsc.subcore_barrier()
    pltpu.sync_copy(accum.at[stripe], out_hbm.at[stripe, pl.ds(cid * H, H)])


def _b_body(x_ref, w1_ref, h1_ref):
    hmat = jnp.dot(x_ref[...], w1_ref[...], preferred_element_type=jnp.float32)
    h1_ref[...] = jnp.concatenate([hmat, jnp.zeros_like(hmat)], axis=1)


def _dinv_block(deg_ref):
    deg = 1.0 + jnp.sum(deg_ref[...], axis=0)
    return lax.rsqrt(deg)[:, None]


def _d_body(p_ref, h1_ref, deg_ref, b1_ref, w2_ref, h2_ref):
    dinv = _dinv_block(deg_ref)
    p0 = p_ref[:, :H]
    p1 = p_ref[:, H:]
    t = (p0 + p1) * dinv + h1_ref[:, :H] * (dinv * dinv)
    o1 = jnp.maximum(t + b1_ref[...], 0.0)
    hmat = jnp.dot(o1, w2_ref[...], preferred_element_type=jnp.float32)
    h2_ref[...] = jnp.concatenate([hmat, jnp.zeros_like(hmat)], axis=1)


def _f_body(q_ref, h2_ref, deg_ref, b2_ref, out_ref):
    dinv = _dinv_block(deg_ref)
    out_ref[...] = ((q_ref[:, :H] + q_ref[:, H:]) * dinv
                    + h2_ref[:, :H] * (dinv * dinv) + b2_ref[...])


def _tc_b(x, W1):
    return pl.pallas_call(
        _b_body,
        grid=(NBLK,),
        in_specs=[
            pl.BlockSpec((MBLK, D), lambda i: (i, 0)),
            pl.BlockSpec((D, H), lambda i: (0, 0)),
        ],
        out_specs=pl.BlockSpec((MBLK, 2 * H), lambda i: (i, 0)),
        out_shape=jax.ShapeDtypeStruct((N_PAD, 2 * H), jnp.float32),
    )(x, W1)


def _tc_d(p, h1, deg, b1, W2):
    return pl.pallas_call(
        _d_body,
        grid=(NBLK,),
        in_specs=[
            pl.BlockSpec((MBLK, 2 * H), lambda i: (i, 0)),
            pl.BlockSpec((MBLK, 2 * H), lambda i: (i, 0)),
            pl.BlockSpec((NC, MBLK), lambda i: (0, i)),
            pl.BlockSpec((1, H), lambda i: (0, 0)),
            pl.BlockSpec((H, H), lambda i: (0, 0)),
        ],
        out_specs=pl.BlockSpec((MBLK, 2 * H), lambda i: (i, 0)),
        out_shape=jax.ShapeDtypeStruct((N_PAD, 2 * H), jnp.float32),
    )(p, h1, deg, b1, W2)


def _tc_f(q, h2, deg, b2):
    return pl.pallas_call(
        _f_body,
        grid=(NBLK,),
        in_specs=[
            pl.BlockSpec((MBLK, 2 * H), lambda i: (i, 0)),
            pl.BlockSpec((MBLK, 2 * H), lambda i: (i, 0)),
            pl.BlockSpec((NC, MBLK), lambda i: (0, i)),
            pl.BlockSpec((1, H), lambda i: (0, 0)),
        ],
        out_specs=pl.BlockSpec((MBLK, H), lambda i: (i, 0)),
        out_shape=jax.ShapeDtypeStruct((N, H), jnp.float32),
    )(q, h2, deg, b2)


def kernel(x, edge_index, w, W1, b1, W2, b2):
    eip = jnp.pad(edge_index,
                  ((0, 0), (0, E_PAD - E))).reshape(2, NW, NCHUNK, K)
    wp = jnp.pad(w, (0, E_PAD - E)).reshape(NW, NCHUNK, K)

    deg = _deg_kernel(eip, wp)
    h1 = _tc_b(x, W1)
    p = _msg_kernel(h1, deg, eip, wp)
    deg2 = deg.reshape(NC, N_PAD)
    h2 = _tc_d(p, h1, deg2, b1.reshape(1, H), W2)
    q = _msg_kernel(h2, deg, eip, wp)
    return _tc_f(q, h2, deg2, b2.reshape(1, H))
